# baseline (speedup 1.0000x reference)
import functools

import jax
import jax.numpy as jnp
from jax import lax
from jax.experimental import pallas as pl
from jax.experimental.pallas import tpu as pltpu

B = 128
FEAT = 2048
DIM = 128
K = 50
N = 4096
M = 0.999
T = 0.07
KN = K * N
SLOTS = B
NBUF = 4
NCHUNK = 7


def _enc_kernel(imq_ref, imk_ref, wq_ref, wk_ref, labels_ref, ct_ref,
                q_ref, k_ref, lpos_ref, laboh_ref, maskf_ref):
  wq = wq_ref[...]
  wk = M * wk_ref[...] + (1.0 - M) * wq
  qraw = jnp.dot(imq_ref[...], wq, preferred_element_type=jnp.float32)
  kraw = jnp.dot(imk_ref[...], wk, preferred_element_type=jnp.float32)
  q = qraw / jnp.sqrt(jnp.sum(qraw * qraw, axis=1, keepdims=True))
  k = kraw / jnp.sqrt(jnp.sum(kraw * kraw, axis=1, keepdims=True))
  q_ref[...] = q
  k_ref[...] = k
  lpos_ref[...] = jnp.sum(q * k, axis=1, keepdims=True) / T
  iota_k = lax.broadcasted_iota(jnp.int32, (B, K), 1)
  laboh = (labels_ref[...] == iota_k).astype(jnp.float32)
  laboh_ref[...] = laboh
  maskf_ref[...] = jnp.dot(laboh, ct_ref[...],
                           preferred_element_type=jnp.float32)


def _sweep_kernel(q_ref, k_ref, lpos_ref, laboh_ref, maskf_ref, qhbm_ref,
                  logits_ref, nqhbm_ref, sexp_ref, stot_ref, slab_ref,
                  carry_ref, qbuf, insems, cpsems):
  c = pl.program_id(0)

  def _copy_in(blk):
    slot = lax.rem(blk, NBUF)
    return pltpu.make_async_copy(
        qhbm_ref.at[:, blk, :], qbuf.at[slot], insems.at[slot])

  def _copy_chunk(t):
    size = 8 if t < NCHUNK - 1 else K - 8 * (NCHUNK - 1)
    return pltpu.make_async_copy(
        qhbm_ref.at[:, pl.ds(8 * t, size), :],
        nqhbm_ref.at[:, pl.ds(8 * t, size), :], cpsems.at[t])

  @pl.when(c == 0)
  def _init():
    sexp_ref[...] = jnp.zeros_like(sexp_ref)
    stot_ref[...] = jnp.zeros_like(stot_ref)
    slab_ref[...] = jnp.zeros_like(slab_ref)
    carry_ref[...] = lpos_ref[...]
    for blk in range(NBUF - 1):
      _copy_in(blk).start()

  for _t in range(NCHUNK):
    @pl.when(c == _t)
    def _start_chunk(_t=_t):
      _copy_chunk(_t).start()

  @pl.when(c == K)
  def _drain():
    for t in range(NCHUNK):
      _copy_chunk(t).wait()

  @pl.when(c < K)
  def _body():
    _copy_in(c).wait()

    @pl.when(c < K - (NBUF - 1))
    def _prefetch():
      _copy_in(c + NBUF - 1).start()

    qb = qbuf[lax.rem(c, NBUF)]
    onehot_c = (lax.broadcasted_iota(jnp.int32, (1, K), 1) == c
                ).astype(jnp.float32)
    ln = jnp.dot(q_ref[...], qb, preferred_element_type=jnp.float32)
    mcol = jnp.sum(maskf_ref[...] * onehot_c, axis=1, keepdims=True)
    lnm = jnp.where(mcol > 0.5, -jnp.inf, ln / T)
    logits_ref[...] = jnp.concatenate(
        [carry_ref[...], lnm[:, :N - 1]], axis=1)
    carry_ref[...] = lnm[:, N - 1:N]
    x = jnp.dot(k_ref[...], qb, preferred_element_type=jnp.float32)
    sexp_ref[...] += jnp.sum(jnp.exp(x / T), axis=1, keepdims=True)
    sx = jnp.sum(x, axis=1, keepdims=True)
    stot_ref[...] += sx
    labcol = jnp.sum(laboh_ref[...] * onehot_c, axis=1, keepdims=True)
    slab_ref[...] += labcol * sx

  @pl.when(c == K)
  def _tail():
    logits_ref[...] = jnp.concatenate(
        [carry_ref[...], jnp.zeros((B, N - 1), jnp.float32)], axis=1)


def _select_kernel(sexp_ref, stot_ref, slab_ref, laboh_ref, k_ref,
                   valsreg_ref, cnt_ref):
  sexp = sexp_ref[...]
  stot = stot_ref[...]
  slab = slab_ref[...]
  e = jnp.float32(2.718281828459045)
  d0 = jnp.float32(N) * (e + jnp.float32(K - 1))
  u = jnp.log(sexp) - ((e - 1.0) * slab + stot) / (d0 * T)
  sel = (u <= jnp.mean(u)).astype(jnp.float32)
  laboh = laboh_ref[...]
  seloh = laboh * sel
  iob = lax.broadcasted_iota(jnp.int32, (B, B), 0)
  job = lax.broadcasted_iota(jnp.int32, (B, B), 1)
  tril = (job < iob).astype(jnp.float32)
  pos_before = jnp.dot(tril, seloh, preferred_element_type=jnp.float32)
  slot = jnp.sum(pos_before * laboh, axis=1, keepdims=True)
  jio = lax.broadcasted_iota(jnp.int32, (B, SLOTS), 1)
  slotmask = (slot.astype(jnp.int32) == jio).astype(jnp.float32) * sel
  kT = jnp.transpose(k_ref[...])
  for c in range(K):
    p_c = slotmask * laboh[:, c:c + 1]
    valsreg_ref[:, c, :] = jnp.dot(kT, p_c,
                                   preferred_element_type=jnp.float32)
  cnt_ref[...] = lax.dot_general(seloh, jnp.ones((B, 1), jnp.float32),
                                 (((0,), (0,)), ((), ())),
                                 preferred_element_type=jnp.float32)


def _region_scatter_kernel(valsreg_ref, cnt_ref, nqin_ref, nqout_ref):
  jio = lax.broadcasted_iota(jnp.int32, (DIM, K, SLOTS), 2)
  cnt3 = cnt_ref[...].astype(jnp.int32)[None]
  nqout_ref[...] = jnp.where(jio < cnt3, valsreg_ref[...], nqin_ref[...])


def _tc_stage(im_q, im_k, labels, W_q, W_k, queue, contras_table):
  labels2d = labels.astype(jnp.int32).reshape(B, 1)
  ct_f32 = contras_table.astype(jnp.float32)

  q, k, lposT, laboh, maskf = pl.pallas_call(
      _enc_kernel,
      out_shape=[
          jax.ShapeDtypeStruct((B, DIM), jnp.float32),
          jax.ShapeDtypeStruct((B, DIM), jnp.float32),
          jax.ShapeDtypeStruct((B, 1), jnp.float32),
          jax.ShapeDtypeStruct((B, K), jnp.float32),
          jax.ShapeDtypeStruct((B, K), jnp.float32),
      ],
  )(im_q, im_k, W_q, W_k, labels2d, ct_f32)

  grid = (K + 1,)
  logits, nq, sexp, stot, slab = pl.pallas_call(
      _sweep_kernel,
      grid=grid,
      in_specs=[
          pl.BlockSpec((B, DIM), lambda c: (0, 0)),
          pl.BlockSpec((B, DIM), lambda c: (0, 0)),
          pl.BlockSpec((B, 1), lambda c: (0, 0)),
          pl.BlockSpec((B, K), lambda c: (0, 0)),
          pl.BlockSpec((B, K), lambda c: (0, 0)),
          pl.BlockSpec(memory_space=pltpu.MemorySpace.HBM),
      ],
      out_specs=[
          pl.BlockSpec((B, N), lambda c: (0, c)),
          pl.BlockSpec(memory_space=pltpu.MemorySpace.HBM),
          pl.BlockSpec((B, 1), lambda c: (0, 0)),
          pl.BlockSpec((B, 1), lambda c: (0, 0)),
          pl.BlockSpec((B, 1), lambda c: (0, 0)),
      ],
      out_shape=[
          jax.ShapeDtypeStruct((B, KN + 1), jnp.float32),
          jax.ShapeDtypeStruct((DIM, K, N), jnp.float32),
          jax.ShapeDtypeStruct((B, 1), jnp.float32),
          jax.ShapeDtypeStruct((B, 1), jnp.float32),
          jax.ShapeDtypeStruct((B, 1), jnp.float32),
      ],
      scratch_shapes=[
          pltpu.VMEM((B, 1), jnp.float32),
          pltpu.VMEM((NBUF, DIM, N), jnp.float32),
          pltpu.SemaphoreType.DMA((NBUF,)),
          pltpu.SemaphoreType.DMA((NCHUNK,)),
      ],
      compiler_params=pltpu.CompilerParams(
          dimension_semantics=("arbitrary",)),
  )(q, k, lposT, laboh, maskf, queue)

  valsreg, cnt = pl.pallas_call(
      _select_kernel,
      out_shape=[
          jax.ShapeDtypeStruct((DIM, K, SLOTS), jnp.float32),
          jax.ShapeDtypeStruct((K, 1), jnp.float32),
      ],
  )(sexp, stot, slab, laboh, k)

  return logits, nq, valsreg, cnt


def kernel(im_q, im_k, labels, W_q, W_k, queue, contras_table):
  logits, nq, valsreg, cnt = _tc_stage(
      im_q, im_k, labels, W_q, W_k, queue, contras_table)

  new_queue = pl.pallas_call(
      _region_scatter_kernel,
      grid=(1,),
      in_specs=[
          pl.BlockSpec((DIM, K, SLOTS), lambda i: (0, 0, 0)),
          pl.BlockSpec((K, 1), lambda i: (0, 0)),
          pl.BlockSpec((DIM, K, SLOTS), lambda i: (0, 0, 0)),
      ],
      out_specs=pl.BlockSpec((DIM, K, SLOTS), lambda i: (0, 0, 0)),
      out_shape=jax.ShapeDtypeStruct((DIM, K, N), jnp.float32),
      input_output_aliases={2: 0},
  )(valsreg, cnt, nq)

  targets = jnp.zeros((B,), dtype=jnp.int32)
  return (logits, targets, new_queue)


# speedup vs baseline: 9.4912x; 9.4912x over previous
import functools

import jax
import jax.numpy as jnp
from jax import lax
from jax.experimental import pallas as pl
from jax.experimental.pallas import tpu as pltpu

B = 128
FEAT = 2048
DIM = 128
K = 50
N = 4096
M = 0.999
T = 0.07
KN = K * N
SLOTS = B
NBUF = 6
PDIST = 3


def _enc_kernel(imq_ref, imk_ref, wq_ref, wk_ref, labels_ref, ct_ref,
                q_ref, k_ref, lpos_ref, laboh_ref, maskf_ref):
  wq = wq_ref[...]
  wk = M * wk_ref[...] + (1.0 - M) * wq
  qraw = jnp.dot(imq_ref[...], wq, preferred_element_type=jnp.float32)
  kraw = jnp.dot(imk_ref[...], wk, preferred_element_type=jnp.float32)
  q = qraw / jnp.sqrt(jnp.sum(qraw * qraw, axis=1, keepdims=True))
  k = kraw / jnp.sqrt(jnp.sum(kraw * kraw, axis=1, keepdims=True))
  q_ref[...] = q
  k_ref[...] = k
  lpos_ref[...] = jnp.sum(q * k, axis=1, keepdims=True) / T
  iota_k = lax.broadcasted_iota(jnp.int32, (B, K), 1)
  laboh = (labels_ref[...] == iota_k).astype(jnp.float32)
  laboh_ref[...] = laboh
  maskf_ref[...] = jnp.dot(laboh, ct_ref[...],
                           preferred_element_type=jnp.float32)


def _sweep_kernel(q_ref, k_ref, lpos_ref, laboh_ref, maskf_ref, qhbm_ref,
                  logits_ref, nqhbm_ref, sexp_ref, stot_ref, slab_ref,
                  carry_ref, qbuf, insems, outsems):
  c = pl.program_id(0)

  def _copy_in(blk):
    slot = lax.rem(blk, NBUF)
    return pltpu.make_async_copy(
        qhbm_ref.at[:, blk, :], qbuf.at[slot], insems.at[slot])

  def _copy_out(blk):
    slot = lax.rem(blk, NBUF)
    return pltpu.make_async_copy(
        qbuf.at[slot], nqhbm_ref.at[:, blk, :], outsems.at[slot])

  @pl.when(c == 0)
  def _init():
    sexp_ref[...] = jnp.zeros_like(sexp_ref)
    stot_ref[...] = jnp.zeros_like(stot_ref)
    slab_ref[...] = jnp.zeros_like(slab_ref)
    carry_ref[...] = lpos_ref[...]
    for blk in range(PDIST):
      _copy_in(blk).start()

  @pl.when(c == K)
  def _drain():
    for t in range(NBUF):
      _copy_out(K - 1 - t).wait()

  @pl.when(c < K)
  def _body():
    _copy_in(c).wait()

    @pl.when(jnp.logical_and(c >= PDIST, c < K - PDIST))
    def _wait_old_out():
      _copy_out(c - PDIST).wait()

    @pl.when(c < K - PDIST)
    def _prefetch():
      _copy_in(c + PDIST).start()

    qb = qbuf[lax.rem(c, NBUF)]
    onehot_c = (lax.broadcasted_iota(jnp.int32, (1, K), 1) == c
                ).astype(jnp.float32)
    ln = jnp.dot(q_ref[...], qb, preferred_element_type=jnp.float32)
    mcol = jnp.sum(maskf_ref[...] * onehot_c, axis=1, keepdims=True)
    lnm = jnp.where(mcol > 0.5, -jnp.inf, ln / T)
    logits_ref[...] = jnp.concatenate(
        [carry_ref[...], lnm[:, :N - 1]], axis=1)
    carry_ref[...] = lnm[:, N - 1:N]
    x = jnp.dot(k_ref[...], qb, preferred_element_type=jnp.float32)
    sexp_ref[...] += jnp.sum(jnp.exp(x / T), axis=1, keepdims=True)
    sx = jnp.sum(x, axis=1, keepdims=True)
    stot_ref[...] += sx
    labcol = jnp.sum(laboh_ref[...] * onehot_c, axis=1, keepdims=True)
    slab_ref[...] += labcol * sx
    _copy_out(c).start()

  @pl.when(c == K)
  def _tail():
    logits_ref[...] = jnp.concatenate(
        [carry_ref[...], jnp.zeros((B, N - 1), jnp.float32)], axis=1)


def _select_kernel(sexp_ref, stot_ref, slab_ref, laboh_ref, k_ref,
                   valsreg_ref, cnt_ref):
  sexp = sexp_ref[...]
  stot = stot_ref[...]
  slab = slab_ref[...]
  e = jnp.float32(2.718281828459045)
  d0 = jnp.float32(N) * (e + jnp.float32(K - 1))
  u = jnp.log(sexp) - ((e - 1.0) * slab + stot) / (d0 * T)
  sel = (u <= jnp.mean(u)).astype(jnp.float32)
  laboh = laboh_ref[...]
  seloh = laboh * sel
  iob = lax.broadcasted_iota(jnp.int32, (B, B), 0)
  job = lax.broadcasted_iota(jnp.int32, (B, B), 1)
  tril = (job < iob).astype(jnp.float32)
  pos_before = jnp.dot(tril, seloh, preferred_element_type=jnp.float32)
  slot = jnp.sum(pos_before * laboh, axis=1, keepdims=True)
  jio = lax.broadcasted_iota(jnp.int32, (B, SLOTS), 1)
  slotmask = (slot.astype(jnp.int32) == jio).astype(jnp.float32) * sel
  kT = jnp.transpose(k_ref[...])
  for c in range(K):
    p_c = slotmask * laboh[:, c:c + 1]
    valsreg_ref[:, c, :] = jnp.dot(kT, p_c,
                                   preferred_element_type=jnp.float32)
  cnt_ref[...] = lax.dot_general(seloh, jnp.ones((B, 1), jnp.float32),
                                 (((0,), (0,)), ((), ())),
                                 preferred_element_type=jnp.float32)


def _region_scatter_kernel(valsreg_ref, cnt_ref, nqin_ref, nqout_ref):
  jio = lax.broadcasted_iota(jnp.int32, (DIM, K, SLOTS), 2)
  cnt3 = cnt_ref[...].astype(jnp.int32)[None]
  nqout_ref[...] = jnp.where(jio < cnt3, valsreg_ref[...], nqin_ref[...])


def _tc_stage(im_q, im_k, labels, W_q, W_k, queue, contras_table):
  labels2d = labels.astype(jnp.int32).reshape(B, 1)
  ct_f32 = contras_table.astype(jnp.float32)

  q, k, lposT, laboh, maskf = pl.pallas_call(
      _enc_kernel,
      out_shape=[
          jax.ShapeDtypeStruct((B, DIM), jnp.float32),
          jax.ShapeDtypeStruct((B, DIM), jnp.float32),
          jax.ShapeDtypeStruct((B, 1), jnp.float32),
          jax.ShapeDtypeStruct((B, K), jnp.float32),
          jax.ShapeDtypeStruct((B, K), jnp.float32),
      ],
  )(im_q, im_k, W_q, W_k, labels2d, ct_f32)

  grid = (K + 1,)
  logits, nq, sexp, stot, slab = pl.pallas_call(
      _sweep_kernel,
      grid=grid,
      in_specs=[
          pl.BlockSpec((B, DIM), lambda c: (0, 0)),
          pl.BlockSpec((B, DIM), lambda c: (0, 0)),
          pl.BlockSpec((B, 1), lambda c: (0, 0)),
          pl.BlockSpec((B, K), lambda c: (0, 0)),
          pl.BlockSpec((B, K), lambda c: (0, 0)),
          pl.BlockSpec(memory_space=pltpu.MemorySpace.HBM),
      ],
      out_specs=[
          pl.BlockSpec((B, N), lambda c: (0, c)),
          pl.BlockSpec(memory_space=pltpu.MemorySpace.HBM),
          pl.BlockSpec((B, 1), lambda c: (0, 0)),
          pl.BlockSpec((B, 1), lambda c: (0, 0)),
          pl.BlockSpec((B, 1), lambda c: (0, 0)),
      ],
      out_shape=[
          jax.ShapeDtypeStruct((B, KN + 1), jnp.float32),
          jax.ShapeDtypeStruct((DIM, K, N), jnp.float32),
          jax.ShapeDtypeStruct((B, 1), jnp.float32),
          jax.ShapeDtypeStruct((B, 1), jnp.float32),
          jax.ShapeDtypeStruct((B, 1), jnp.float32),
      ],
      scratch_shapes=[
          pltpu.VMEM((B, 1), jnp.float32),
          pltpu.VMEM((NBUF, DIM, N), jnp.float32),
          pltpu.SemaphoreType.DMA((NBUF,)),
          pltpu.SemaphoreType.DMA((NBUF,)),
      ],
      compiler_params=pltpu.CompilerParams(
          dimension_semantics=("arbitrary",)),
  )(q, k, lposT, laboh, maskf, queue)

  valsreg, cnt = pl.pallas_call(
      _select_kernel,
      out_shape=[
          jax.ShapeDtypeStruct((DIM, K, SLOTS), jnp.float32),
          jax.ShapeDtypeStruct((K, 1), jnp.float32),
      ],
  )(sexp, stot, slab, laboh, k)

  return logits, nq, valsreg, cnt


def kernel(im_q, im_k, labels, W_q, W_k, queue, contras_table):
  logits, nq, valsreg, cnt = _tc_stage(
      im_q, im_k, labels, W_q, W_k, queue, contras_table)

  new_queue = pl.pallas_call(
      _region_scatter_kernel,
      grid=(1,),
      in_specs=[
          pl.BlockSpec((DIM, K, SLOTS), lambda i: (0, 0, 0)),
          pl.BlockSpec((K, 1), lambda i: (0, 0)),
          pl.BlockSpec((DIM, K, SLOTS), lambda i: (0, 0, 0)),
      ],
      out_specs=pl.BlockSpec((DIM, K, SLOTS), lambda i: (0, 0, 0)),
      out_shape=jax.ShapeDtypeStruct((DIM, K, N), jnp.float32),
      input_output_aliases={2: 0},
  )(valsreg, cnt, nq)

  targets = jnp.zeros((B,), dtype=jnp.int32)
  return (logits, targets, new_queue)
